# Initial kernel scaffold; baseline (speedup 1.0000x reference)
import jax
import jax.numpy as jnp
from jax.experimental import pallas as pl


def kernel(k_new, v_new, cos, sin, cache_k, cache_v, positions):
    raise NotImplementedError("write your pallas kernel here")


# speedup vs baseline: 2.0286x; 2.0286x over previous
import functools

import jax
import jax.numpy as jnp
from jax.experimental import pallas as pl
from jax.experimental.pallas import tpu as pltpu


def _copy_body(ck_ref, cv_ref, out_ref):
    out_ref[0, 0] = ck_ref[0]
    out_ref[1, 0] = cv_ref[0]


def _scatter_body(outin_ref, knew_ref, vnew_ref, cosf_ref, sina_ref,
                  pos_ref, out_ref, cosbuf, sinbuf, rotbuf, sem):
    del outin_ref
    b = pl.program_id(0)
    u, h, d = rotbuf.shape
    base = pos_ref[b, 0]
    cpc = pltpu.make_async_copy(cosf_ref.at[pl.ds(base, u)], cosbuf, sem.at[0])
    cpc.start()
    cps = pltpu.make_async_copy(sina_ref.at[pl.ds(base, u)], sinbuf, sem.at[1])
    cps.start()
    cpv = pltpu.make_async_copy(
        vnew_ref.at[0], out_ref.at[1, b, pl.ds(base, u)], sem.at[2])
    cpv.start()
    cpc.wait()
    cps.wait()
    x = knew_ref[0]
    xp = pltpu.roll(x, d - 1, 2)
    xm = pltpu.roll(x, 1, 2)
    lane = jax.lax.broadcasted_iota(jnp.int32, x.shape, 2)
    even = (lane % 2) == 0
    c = cosbuf[...]
    s = sinbuf[...]
    rotbuf[...] = x * c + jnp.where(even, xp, xm) * s
    cpk = pltpu.make_async_copy(
        rotbuf, out_ref.at[0, b, pl.ds(base, u)], sem.at[3])
    cpk.start()
    cpk.wait()
    cpv.wait()


@functools.partial(jax.jit, static_argnames=("interpret",))
def _run(k_new, v_new, cos, sin, cache_k, cache_v, positions, interpret=False):
    b, u, h, d = k_new.shape
    cl = cache_k.shape[1]
    half = d // 2
    f32 = jnp.float32

    cosf = jnp.repeat(cos, 2, axis=1).reshape(cl, 1, d)
    sgn = jnp.tile(jnp.array([-1.0, 1.0], dtype=f32), half)
    sina = (jnp.repeat(sin, 2, axis=1) * sgn[None, :]).reshape(cl, 1, d)

    t_blk = 256
    s_steps = cl // t_blk
    out1 = pl.pallas_call(
        _copy_body,
        grid=(b, s_steps),
        in_specs=[
            pl.BlockSpec((1, t_blk, h, d), lambda i, s: (i, s, 0, 0)),
            pl.BlockSpec((1, t_blk, h, d), lambda i, s: (i, s, 0, 0)),
        ],
        out_specs=pl.BlockSpec((2, 1, t_blk, h, d),
                               lambda i, s: (0, i, s, 0, 0)),
        out_shape=jax.ShapeDtypeStruct((2, b, cl, h, d), f32),
        interpret=interpret,
    )(cache_k, cache_v)

    out = pl.pallas_call(
        _scatter_body,
        grid=(b,),
        in_specs=[
            pl.BlockSpec(memory_space=pl.ANY),
            pl.BlockSpec((1, u, h, d), lambda i: (i, 0, 0, 0)),
            pl.BlockSpec((1, u, h, d), lambda i: (i, 0, 0, 0)),
            pl.BlockSpec(memory_space=pl.ANY),
            pl.BlockSpec(memory_space=pl.ANY),
            pl.BlockSpec(memory_space=pltpu.SMEM),
        ],
        out_specs=pl.BlockSpec(memory_space=pl.ANY),
        out_shape=jax.ShapeDtypeStruct((2, b, cl, h, d), f32),
        scratch_shapes=[
            pltpu.VMEM((u, 1, d), f32),
            pltpu.VMEM((u, 1, d), f32),
            pltpu.VMEM((u, h, d), f32),
            pltpu.SemaphoreType.DMA((4,)),
        ],
        input_output_aliases={0: 0},
        interpret=interpret,
    )(out1, k_new, v_new, cosf, sina, positions)

    return out


def kernel(k_new, v_new, cos, sin, cache_k, cache_v, positions):
    return _run(k_new, v_new, cos, sin, cache_k, cache_v, positions)
